# Initial kernel scaffold; baseline (speedup 1.0000x reference)
import jax
import jax.numpy as jnp
from jax.experimental import pallas as pl


def kernel(unknown, known2, feats2, known3, feats3, known4, feats4, match_points, W_fc, W_cls):
    raise NotImplementedError("write your pallas kernel here")


# speedup vs baseline: 7.2971x; 7.2971x over previous
import jax
import jax.numpy as jnp
from jax.experimental import pallas as pl

_R = 128
_BIG = 3.0e38


def _body(ub_ref, a2_ref, k2_ref, b2_2_ref, f2_ref, k3_ref, b2_3_ref,
          f3_ref, k4_ref, b2_4_ref, f4_ref, mb_ref, b2_m_ref,
          wfc_ref, wcls_ref, pred_ref, gt_ref):
    ub = ub_ref[...]
    a2 = a2_ref[...]

    def sqdist(kbT, b2):
        dots = jax.lax.dot_general(
            ub, kbT, (((1,), (0,)), ((), ())),
            preferred_element_type=jnp.float32)
        return jnp.maximum((a2 + b2) - 2.0 * dots, 0.0)

    def interp(kbT, b2, feats):
        d2 = sqdist(kbT, b2)
        m = kbT.shape[1]
        iota = jax.lax.broadcasted_iota(jnp.int32, (_R, m), 1)
        cur = d2
        sels = []
        dists = []
        for _ in range(3):
            mn = jnp.min(cur, axis=1, keepdims=True)
            idx = jnp.min(jnp.where(cur == mn, iota, m), axis=1,
                          keepdims=True)
            sel = iota == idx
            cur = jnp.where(sel, _BIG, cur)
            sels.append(sel)
            dists.append(jnp.sqrt(mn))
        r0 = 1.0 / (dists[0] + 1e-8)
        r1 = 1.0 / (dists[1] + 1e-8)
        r2 = 1.0 / (dists[2] + 1e-8)
        norm = r0 + r1 + r2
        w_hot = (jnp.where(sels[0], r0 / norm, 0.0)
                 + jnp.where(sels[1], r1 / norm, 0.0)
                 + jnp.where(sels[2], r2 / norm, 0.0))
        return jax.lax.dot_general(
            w_hot, feats, (((1,), (0,)), ((), ())),
            precision=jax.lax.Precision.HIGHEST,
            preferred_element_type=jnp.float32)

    p0 = interp(k2_ref[...], b2_2_ref[...], f2_ref[...])
    p1 = interp(k3_ref[...], b2_3_ref[...], f3_ref[...])
    p2 = interp(k4_ref[...], b2_4_ref[...], f4_ref[...])
    x96 = jnp.concatenate([p0, p1, p2], axis=1)
    pointwise = jax.lax.dot_general(
        x96.astype(jnp.bfloat16), wfc_ref[...],
        (((1,), (1,)), ((), ())), preferred_element_type=jnp.float32)
    pred8 = jax.lax.dot_general(
        pointwise.astype(jnp.bfloat16), wcls_ref[...],
        (((1,), (1,)), ((), ())), preferred_element_type=jnp.float32)
    pred_ref[...] = pred8[:, 0:1]

    d2m = sqdist(mb_ref[...], b2_m_ref[...])
    mn = jnp.min(d2m, axis=1, keepdims=True)
    gt_ref[...] = (jnp.sqrt(mn) < 0.5).astype(jnp.float32)


def kernel(unknown, known2, feats2, known3, feats3, known4, feats4,
           match_points, W_fc, W_cls):
    n = unknown.shape[0]
    grid = (n // _R,)

    def row_norm(x):
        return jnp.sum(x * x, axis=1, keepdims=True).T

    bf = jnp.bfloat16
    full = lambda shape: pl.BlockSpec(shape, lambda i: (0, 0))
    pred, gt = pl.pallas_call(
        _body,
        grid=grid,
        in_specs=[
            pl.BlockSpec((_R, 3), lambda i: (i, 0)),
            pl.BlockSpec((_R, 1), lambda i: (i, 0)),
            full((3, 8192)), full((1, 8192)), full((8192, 32)),
            full((3, 4096)), full((1, 4096)), full((4096, 32)),
            full((3, 2048)), full((1, 2048)), full((2048, 32)),
            full((3, 2048)), full((1, 2048)),
            full((64, 96)), full((8, 64)),
        ],
        out_specs=[
            pl.BlockSpec((_R, 1), lambda i: (i, 0)),
            pl.BlockSpec((_R, 1), lambda i: (i, 0)),
        ],
        out_shape=[
            jax.ShapeDtypeStruct((n, 1), jnp.float32),
            jax.ShapeDtypeStruct((n, 1), jnp.float32),
        ],
    )(unknown.astype(bf),
      jnp.sum(unknown * unknown, axis=1, keepdims=True),
      known2.astype(bf).T, row_norm(known2), feats2,
      known3.astype(bf).T, row_norm(known3), feats3,
      known4.astype(bf).T, row_norm(known4), feats4,
      match_points.astype(bf).T, row_norm(match_points),
      W_fc.astype(bf),
      jnp.concatenate([W_cls, jnp.zeros((7, 64), W_cls.dtype)], 0).astype(bf))
    return pred, gt.reshape(n)


# speedup vs baseline: 12.7923x; 1.7531x over previous
import functools

import jax
import jax.numpy as jnp
from jax import lax
from jax.experimental import pallas as pl
from jax.experimental.pallas import tpu as pltpu
from jax.experimental.pallas import tpu_sc as plsc

_R = 128
_BIG = 3.0e38

_NW = 32
_CHUNK = 128


def _nn_body(ub_ref, a2_ref, k2_ref, b2_2_ref, k3_ref, b2_3_ref,
             k4_ref, b2_4_ref, mb_ref, b2_m_ref,
             idx2_ref, w2_ref, idx3_ref, w3_ref, idx4_ref, w4_ref,
             gt_ref):
    ub = ub_ref[...]
    a2 = a2_ref[...]

    def sqdist(kbT, b2):
        dots = jax.lax.dot_general(
            ub, kbT, (((1,), (0,)), ((), ())),
            preferred_element_type=jnp.float32)
        return jnp.maximum((a2 + b2) - 2.0 * dots, 0.0)

    def top3(kbT, b2, idx_ref, w_ref):
        d2 = sqdist(kbT, b2)
        m = kbT.shape[1]
        iota = lax.broadcasted_iota(jnp.int32, (_R, m), 1)
        cur = d2
        idxs = []
        dists = []
        for _ in range(3):
            mn = jnp.min(cur, axis=1, keepdims=True)
            idx = jnp.min(jnp.where(cur == mn, iota, m), axis=1,
                          keepdims=True)
            cur = jnp.where(iota == idx, _BIG, cur)
            idxs.append(idx)
            dists.append(jnp.sqrt(mn))
        r0 = 1.0 / (dists[0] + 1e-8)
        r1 = 1.0 / (dists[1] + 1e-8)
        r2 = 1.0 / (dists[2] + 1e-8)
        norm = r0 + r1 + r2
        idx_ref[...] = jnp.concatenate(idxs, axis=1)
        w_ref[...] = jnp.concatenate(
            [r0 / norm, r1 / norm, r2 / norm], axis=1)

    top3(k2_ref[...], b2_2_ref[...], idx2_ref, w2_ref)
    top3(k3_ref[...], b2_3_ref[...], idx3_ref, w3_ref)
    top3(k4_ref[...], b2_4_ref[...], idx4_ref, w4_ref)

    d2m = sqdist(mb_ref[...], b2_m_ref[...])
    mn = jnp.min(d2m, axis=1, keepdims=True)
    gt_ref[...] = (jnp.sqrt(mn) < 0.5).astype(jnp.float32)


def _combine_body(g2_ref, w2_ref, g3_ref, w3_ref, g4_ref, w4_ref,
                  wfc_ref, wcls_ref, pred_ref):
    def part(g_ref, w_ref):
        return ((g_ref[0] * w_ref[:, 0:1]
                 + g_ref[1] * w_ref[:, 1:2])
                + g_ref[2] * w_ref[:, 2:3])[:, 0:32]

    x96 = jnp.concatenate(
        [part(g2_ref, w2_ref), part(g3_ref, w3_ref),
         part(g4_ref, w4_ref)], axis=1)
    pointwise = jax.lax.dot_general(
        x96.astype(jnp.bfloat16), wfc_ref[...],
        (((1,), (1,)), ((), ())), preferred_element_type=jnp.float32)
    pred8 = jax.lax.dot_general(
        pointwise.astype(jnp.bfloat16), wcls_ref[...],
        (((1,), (1,)), ((), ())), preferred_element_type=jnp.float32)
    pred_ref[...] = pred8[:, 0:1]


def _sc_gather(idx2_h, f2_h, idx3_h, f3_h, idx4_h, f4_h,
               out2, out3, out4, idx_v, rows_v, sem):
    wid = lax.axis_index("s") * 2 + lax.axis_index("c")
    nchunk = idx_v.shape[0]
    rows_per_w = nchunk * _CHUNK
    base = wid * rows_per_w
    for idx_h, tab, out in ((idx2_h, f2_h, out2), (idx3_h, f3_h, out3),
                            (idx4_h, f4_h, out4)):
        pltpu.sync_copy(idx_h.at[wid], idx_v)
        for j in range(nchunk):
            pltpu.async_copy(tab.at[idx_v.at[j]],
                             rows_v.at[pl.ds(j * _CHUNK, _CHUNK)],
                             sem).wait()
        pltpu.sync_copy(rows_v, out.at[pl.ds(base, rows_per_w)])


def kernel(unknown, known2, feats2, known3, feats3, known4, feats4,
           match_points, W_fc, W_cls):
    n = unknown.shape[0]
    nk = 3 * n
    rows_per_w = nk // _NW
    nchunk = rows_per_w // _CHUNK

    def row_norm(x):
        return jnp.sum(x * x, axis=1, keepdims=True).T

    bf = jnp.bfloat16
    full = lambda shape: pl.BlockSpec(shape, lambda i: (0, 0))
    rblk = lambda w: pl.BlockSpec((_R, w), lambda i: (i, 0))
    idx2, w2, idx3, w3, idx4, w4, gt = pl.pallas_call(
        _nn_body,
        grid=(n // _R,),
        in_specs=[
            rblk(3), rblk(1),
            full((3, 8192)), full((1, 8192)),
            full((3, 4096)), full((1, 4096)),
            full((3, 2048)), full((1, 2048)),
            full((3, 2048)), full((1, 2048)),
        ],
        out_specs=[rblk(3), rblk(3), rblk(3), rblk(3), rblk(3), rblk(3),
                   rblk(1)],
        out_shape=[
            jax.ShapeDtypeStruct((n, 3), jnp.int32),
            jax.ShapeDtypeStruct((n, 3), jnp.float32),
            jax.ShapeDtypeStruct((n, 3), jnp.int32),
            jax.ShapeDtypeStruct((n, 3), jnp.float32),
            jax.ShapeDtypeStruct((n, 3), jnp.int32),
            jax.ShapeDtypeStruct((n, 3), jnp.float32),
            jax.ShapeDtypeStruct((n, 1), jnp.float32),
        ],
    )(unknown.astype(bf),
      jnp.sum(unknown * unknown, axis=1, keepdims=True),
      known2.astype(bf).T, row_norm(known2),
      known3.astype(bf).T, row_norm(known3),
      known4.astype(bf).T, row_norm(known4),
      match_points.astype(bf).T, row_norm(match_points))

    def sc_idx(idx):
        return idx.T.reshape(_NW, nchunk, _CHUNK)

    gather = pl.kernel(
        _sc_gather,
        mesh=plsc.VectorSubcoreMesh(core_axis_name="c",
                                    subcore_axis_name="s"),
        out_type=[jax.ShapeDtypeStruct((nk, 128), jnp.float32)] * 3,
        scratch_types=[
            pltpu.VMEM((nchunk, _CHUNK), jnp.int32),
            pltpu.VMEM((rows_per_w, 128), jnp.float32),
            pltpu.SemaphoreType.DMA,
        ],
    )

    def pad128(f):
        return jnp.pad(f, ((0, 0), (0, 96)))

    g2, g3, g4 = gather(sc_idx(idx2), pad128(feats2),
                        sc_idx(idx3), pad128(feats3),
                        sc_idx(idx4), pad128(feats4))

    rc = 1024
    gblk = pl.BlockSpec((3, rc, 128), lambda i: (0, i, 0))
    wblk = pl.BlockSpec((rc, 3), lambda i: (i, 0))
    pred = pl.pallas_call(
        _combine_body,
        grid=(n // rc,),
        in_specs=[
            gblk, wblk, gblk, wblk, gblk, wblk,
            full((64, 96)), full((8, 64)),
        ],
        out_specs=[pl.BlockSpec((rc, 1), lambda i: (i, 0))],
        out_shape=[jax.ShapeDtypeStruct((n, 1), jnp.float32)],
    )(g2.reshape(3, n, 128), w2, g3.reshape(3, n, 128), w3,
      g4.reshape(3, n, 128), w4,
      W_fc.astype(bf),
      jnp.concatenate([W_cls, jnp.zeros((7, 64), W_cls.dtype)], 0).astype(bf))[0]
    return pred, gt.reshape(n)


# speedup vs baseline: 15.0482x; 1.1763x over previous
import jax
import jax.numpy as jnp
from jax import lax
from jax.experimental import pallas as pl
from jax.experimental.pallas import tpu as pltpu
from jax.experimental.pallas import tpu_sc as plsc

_R = 256
_BIG = 3.0e38

_NW = 32
_CHUNK = 128


def _nn_body(ub_ref, a2_ref, k2_ref, b2_2_ref, k3_ref, b2_3_ref,
             k4_ref, b2_4_ref, mb_ref, b2_m_ref,
             idx2_ref, w2_ref, idx3_ref, w3_ref, idx4_ref, w4_ref,
             gt_ref):
    ub = ub_ref[...]
    a2 = a2_ref[...]

    def sqdist(kbT, b2):
        dots = jax.lax.dot_general(
            ub, kbT, (((1,), (0,)), ((), ())),
            preferred_element_type=jnp.float32)
        return jnp.maximum((a2 + b2) - 2.0 * dots, 0.0)

    def top3(kbT, b2, idx_ref, w_ref):
        d2 = sqdist(kbT, b2)
        m = kbT.shape[1]
        big = jnp.full((_R, 128), _BIG, jnp.float32)
        zero = jnp.zeros((_R, 128), jnp.int32)
        a0, a1, a2 = big, big, big
        i0, i1, i2 = zero, zero, zero
        for g in range(m // 128):
            x = d2[:, 128 * g:128 * (g + 1)]
            c0 = x < a0
            c1 = x < a1
            c2 = x < a2
            na1 = jnp.where(c0, a0, jnp.where(c1, x, a1))
            ni1 = jnp.where(c0, i0, jnp.where(c1, g, i1))
            na2 = jnp.where(c1, a1, jnp.where(c2, x, a2))
            ni2 = jnp.where(c1, i1, jnp.where(c2, g, i2))
            a0 = jnp.minimum(a0, x)
            i0 = jnp.where(c0, g, i0)
            a1, i1, a2, i2 = na1, ni1, na2, ni2
        lane = lax.broadcasted_iota(jnp.int32, (_R, 128), 1)
        c0 = i0 * 128 + lane
        c1 = i1 * 128 + lane
        c2 = i2 * 128 + lane
        bigi = jnp.int32(1 << 30)
        idxs = []
        dists = []
        for _ in range(3):
            mn = jnp.min(a0, axis=1, keepdims=True)
            idx = jnp.min(jnp.where(a0 == mn, c0, bigi), axis=1,
                          keepdims=True)
            rem = c0 == idx
            a0 = jnp.where(rem, a1, a0)
            c0 = jnp.where(rem, c1, c0)
            a1 = jnp.where(rem, a2, a1)
            c1 = jnp.where(rem, c2, c1)
            a2 = jnp.where(rem, _BIG, a2)
            idxs.append(idx)
            dists.append(jnp.sqrt(mn))
        r0 = 1.0 / (dists[0] + 1e-8)
        r1 = 1.0 / (dists[1] + 1e-8)
        r2 = 1.0 / (dists[2] + 1e-8)
        norm = r0 + r1 + r2
        idx_ref[...] = jnp.concatenate(idxs, axis=1)
        w_ref[...] = jnp.concatenate(
            [r0 / norm, r1 / norm, r2 / norm], axis=1)

    top3(k2_ref[...], b2_2_ref[...], idx2_ref, w2_ref)
    top3(k3_ref[...], b2_3_ref[...], idx3_ref, w3_ref)
    top3(k4_ref[...], b2_4_ref[...], idx4_ref, w4_ref)

    d2m = sqdist(mb_ref[...], b2_m_ref[...])
    mn = jnp.min(d2m, axis=1, keepdims=True)
    gt_ref[...] = (jnp.sqrt(mn) < 0.5).astype(jnp.float32)


def _combine_body(g2_ref, w2_ref, g3_ref, w3_ref, g4_ref, w4_ref,
                  wfc_ref, wcls_ref, pred_ref):
    def part(g_ref, w_ref):
        return ((g_ref[0] * w_ref[:, 0:1]
                 + g_ref[1] * w_ref[:, 1:2])
                + g_ref[2] * w_ref[:, 2:3])[:, 0:32]

    x96 = jnp.concatenate(
        [part(g2_ref, w2_ref), part(g3_ref, w3_ref),
         part(g4_ref, w4_ref)], axis=1)
    pointwise = jax.lax.dot_general(
        x96.astype(jnp.bfloat16), wfc_ref[...],
        (((1,), (1,)), ((), ())), preferred_element_type=jnp.float32)
    pred8 = jax.lax.dot_general(
        pointwise.astype(jnp.bfloat16), wcls_ref[...],
        (((1,), (1,)), ((), ())), preferred_element_type=jnp.float32)
    pred_ref[...] = pred8[:, 0:1]


def _sc_gather(idx2_h, f2_h, idx3_h, f3_h, idx4_h, f4_h,
               out2, out3, out4, idx_v, rows_v, sem):
    wid = lax.axis_index("s") * 2 + lax.axis_index("c")
    nchunk = idx_v.shape[0]
    rows_per_w = nchunk * _CHUNK
    base = wid * rows_per_w
    for idx_h, tab, out in ((idx2_h, f2_h, out2), (idx3_h, f3_h, out3),
                            (idx4_h, f4_h, out4)):
        pltpu.sync_copy(idx_h.at[wid], idx_v)
        copies = [pltpu.async_copy(tab.at[idx_v.at[j]],
                                   rows_v.at[pl.ds(j * _CHUNK, _CHUNK)],
                                   sem)
                  for j in range(nchunk)]
        for c in copies:
            c.wait()
        pltpu.sync_copy(rows_v, out.at[pl.ds(base, rows_per_w)])


def kernel(unknown, known2, feats2, known3, feats3, known4, feats4,
           match_points, W_fc, W_cls):
    n = unknown.shape[0]
    nk = 3 * n
    rows_per_w = nk // _NW
    nchunk = rows_per_w // _CHUNK

    def row_norm(x):
        return jnp.sum(x * x, axis=1, keepdims=True).T

    bf = jnp.bfloat16
    full = lambda shape: pl.BlockSpec(shape, lambda i: (0, 0))
    rblk = lambda w: pl.BlockSpec((_R, w), lambda i: (i, 0))
    idx2, w2, idx3, w3, idx4, w4, gt = pl.pallas_call(
        _nn_body,
        grid=(n // _R,),
        in_specs=[
            rblk(3), rblk(1),
            full((3, 8192)), full((1, 8192)),
            full((3, 4096)), full((1, 4096)),
            full((3, 2048)), full((1, 2048)),
            full((3, 2048)), full((1, 2048)),
        ],
        out_specs=[rblk(3), rblk(3), rblk(3), rblk(3), rblk(3), rblk(3),
                   rblk(1)],
        out_shape=[
            jax.ShapeDtypeStruct((n, 3), jnp.int32),
            jax.ShapeDtypeStruct((n, 3), jnp.float32),
            jax.ShapeDtypeStruct((n, 3), jnp.int32),
            jax.ShapeDtypeStruct((n, 3), jnp.float32),
            jax.ShapeDtypeStruct((n, 3), jnp.int32),
            jax.ShapeDtypeStruct((n, 3), jnp.float32),
            jax.ShapeDtypeStruct((n, 1), jnp.float32),
        ],
    )(unknown.astype(bf),
      jnp.sum(unknown * unknown, axis=1, keepdims=True),
      known2.astype(bf).T, row_norm(known2),
      known3.astype(bf).T, row_norm(known3),
      known4.astype(bf).T, row_norm(known4),
      match_points.astype(bf).T, row_norm(match_points))

    def sc_idx(idx):
        return idx.T.reshape(_NW, nchunk, _CHUNK)

    gather = pl.kernel(
        _sc_gather,
        mesh=plsc.VectorSubcoreMesh(core_axis_name="c",
                                    subcore_axis_name="s"),
        out_type=[jax.ShapeDtypeStruct((nk, 128), jnp.float32)] * 3,
        scratch_types=[
            pltpu.VMEM((nchunk, _CHUNK), jnp.int32),
            pltpu.VMEM((rows_per_w, 128), jnp.float32),
            pltpu.SemaphoreType.DMA,
        ],
    )

    def pad128(f):
        return jnp.pad(f, ((0, 0), (0, 96)))

    g2, g3, g4 = gather(sc_idx(idx2), pad128(feats2),
                        sc_idx(idx3), pad128(feats3),
                        sc_idx(idx4), pad128(feats4))

    rc = 1024
    gblk = pl.BlockSpec((3, rc, 128), lambda i: (0, i, 0))
    wblk = pl.BlockSpec((rc, 3), lambda i: (i, 0))
    pred = pl.pallas_call(
        _combine_body,
        grid=(n // rc,),
        in_specs=[
            gblk, wblk, gblk, wblk, gblk, wblk,
            full((64, 96)), full((8, 64)),
        ],
        out_specs=[pl.BlockSpec((rc, 1), lambda i: (i, 0))],
        out_shape=[jax.ShapeDtypeStruct((n, 1), jnp.float32)],
    )(g2.reshape(3, n, 128), w2, g3.reshape(3, n, 128), w3,
      g4.reshape(3, n, 128), w4,
      W_fc.astype(bf),
      jnp.concatenate([W_cls, jnp.zeros((7, 64), W_cls.dtype)], 0).astype(bf))[0]
    return pred, gt.reshape(n)
